# baseline (speedup 1.0000x reference)
import functools

import jax
import jax.numpy as jnp
from jax import lax
from jax.experimental import pallas as pl
from jax.experimental.pallas import tpu as pltpu
from jax.experimental.pallas import tpu_sc as plsc


def _ceil_to(n, m):
    return (n + m - 1) // m * m


_SC_NC = 2
_SC_NS = 16
_SC_CH = 128


def _sc_gather(table, idx):
    n, d = table.shape
    e = idx.shape[0]
    nw = _SC_NC * _SC_NS
    bpw = e // nw
    nch = bpw // _SC_CH
    mesh = plsc.VectorSubcoreMesh(core_axis_name="c", subcore_axis_name="s")

    @functools.partial(
        pl.kernel, mesh=mesh,
        out_type=jax.ShapeDtypeStruct((e, d), jnp.float32),
        scratch_types=[
            pltpu.VMEM((_SC_CH,), jnp.int32),
            pltpu.VMEM((_SC_CH, d), jnp.float32),
            pltpu.SemaphoreType.DMA,
        ],
    )
    def gather_k(table_hbm, idx_hbm, out_hbm, idx_v, rows_v, sem):
        wid = lax.axis_index("s") * _SC_NC + lax.axis_index("c")
        base = wid * bpw

        def body(j, carry):
            st = base + j * _SC_CH
            pltpu.sync_copy(idx_hbm.at[pl.ds(st, _SC_CH)], idx_v)
            pltpu.async_copy(table_hbm.at[idx_v], rows_v, sem).wait()
            pltpu.sync_copy(rows_v, out_hbm.at[pl.ds(st, _SC_CH)])
            return carry

        lax.fori_loop(0, nch, body, 0)

    return gather_k(table, idx)


def _sc_gather2(table_l, idx_l, table_r, idx_r):
    n, d = table_l.shape
    e = idx_l.shape[0]
    nw = _SC_NC * _SC_NS
    bpw = e // nw
    nch = bpw // _SC_CH
    mesh = plsc.VectorSubcoreMesh(core_axis_name="c", subcore_axis_name="s")

    @functools.partial(
        pl.kernel, mesh=mesh,
        out_type=(jax.ShapeDtypeStruct((e, d), jnp.float32),
                  jax.ShapeDtypeStruct((e, d), jnp.float32)),
        scratch_types=[
            pltpu.VMEM((_SC_CH,), jnp.int32),
            pltpu.VMEM((_SC_CH,), jnp.int32),
            pltpu.VMEM((_SC_CH, d), jnp.float32),
            pltpu.VMEM((_SC_CH, d), jnp.float32),
            pltpu.SemaphoreType.DMA,
            pltpu.SemaphoreType.DMA,
        ],
    )
    def gather_k(tl_hbm, il_hbm, tr_hbm, ir_hbm, ol_hbm, or_hbm,
                 il_v, ir_v, rl_v, rr_v, sem_l, sem_r):
        wid = lax.axis_index("s") * _SC_NC + lax.axis_index("c")
        base = wid * bpw

        def body(j, carry):
            st = base + j * _SC_CH
            pltpu.sync_copy(il_hbm.at[pl.ds(st, _SC_CH)], il_v)
            pltpu.sync_copy(ir_hbm.at[pl.ds(st, _SC_CH)], ir_v)
            cl = pltpu.async_copy(tl_hbm.at[il_v], rl_v, sem_l)
            cr = pltpu.async_copy(tr_hbm.at[ir_v], rr_v, sem_r)
            cl.wait()
            pltpu.sync_copy(rl_v, ol_hbm.at[pl.ds(st, _SC_CH)])
            cr.wait()
            pltpu.sync_copy(rr_v, or_hbm.at[pl.ds(st, _SC_CH)])
            return carry

        lax.fori_loop(0, nch, body, 0)

    return gather_k(table_l, idx_l, table_r, idx_r)


def _sc_scatter(msg3, dst, nd_pad):
    nch, epad, w = msg3.shape
    nch2 = nch // _SC_NC
    ebt = epad // _SC_NS
    nech = ebt // _SC_CH
    rpt = nd_pad // _SC_NS
    zin = jnp.zeros((nd_pad, w), jnp.float32)
    mesh = plsc.VectorSubcoreMesh(core_axis_name="c", subcore_axis_name="s")

    @functools.partial(
        pl.kernel, mesh=mesh,
        out_type=jax.ShapeDtypeStruct((nch * nd_pad, w), jnp.float32),
        scratch_types=[
            pltpu.VMEM((_SC_CH,), jnp.int32),
            pltpu.VMEM((_SC_CH, w), jnp.float32),
            pltpu.VMEM_SHARED((nd_pad, w), jnp.float32),
        ],
    )
    def scat_k(msg_hbm, dst_hbm, zin_hbm, out_hbm, idx_v, rows_v, acc_sh):
        cid = lax.axis_index("c")
        sid = lax.axis_index("s")
        nrch = rpt // _SC_CH
        for c_local in range(nch2):
            c = cid * nch2 + c_local

            def zbody(k, carry):
                rs = sid * rpt + k * _SC_CH
                pltpu.sync_copy(zin_hbm.at[pl.ds(rs, _SC_CH)], rows_v)
                pltpu.sync_copy(rows_v, acc_sh.at[pl.ds(rs, _SC_CH)])
                return carry

            lax.fori_loop(0, nrch, zbody, 0)
            plsc.subcore_barrier()

            def body(j, carry):
                est = c * epad + sid * ebt + j * _SC_CH
                pltpu.sync_copy(dst_hbm.at[pl.ds(sid * ebt + j * _SC_CH,
                                                 _SC_CH)], idx_v)
                pltpu.sync_copy(msg_hbm.at[pl.ds(est, _SC_CH)], rows_v)
                pltpu.sync_copy(rows_v, acc_sh.at[idx_v], add=True)
                return carry

            lax.fori_loop(0, nech, body, 0)
            plsc.subcore_barrier()

            def wbody(k, carry):
                rs = sid * rpt + k * _SC_CH
                pltpu.sync_copy(acc_sh.at[pl.ds(rs, _SC_CH)], rows_v)
                pltpu.sync_copy(rows_v,
                                out_hbm.at[pl.ds(c * nd_pad + rs, _SC_CH)])
                return carry

            lax.fori_loop(0, nrch, wbody, 0)
            plsc.subcore_barrier()

    return scat_k(msg3.reshape(nch * epad, w), dst, zin).reshape(
        nch, nd_pad, w)


def _linear_body(x_ref, w_ref, b_ref, o_ref, *, relu_in, relu_out):
    x = x_ref[...]
    if relu_in:
        x = jnp.maximum(x, 0.0)
    y = jnp.dot(x, w_ref[...], preferred_element_type=jnp.float32) + b_ref[...]
    if relu_out:
        y = jnp.maximum(y, 0.0)
    o_ref[...] = y


def _linear(x, w, b, relu_in=False, relu_out=False, block=1024):
    n, kin = x.shape
    kout = w.shape[1]
    if kin < 8:
        x = jnp.pad(x, ((0, 0), (0, 8 - kin)))
        w = jnp.pad(w, ((0, 8 - kin), (0, 0)))
        kin = 8
    npad = _ceil_to(n, block)
    if npad != n:
        x = jnp.pad(x, ((0, npad - n), (0, 0)))
    out = pl.pallas_call(
        functools.partial(_linear_body, relu_in=relu_in, relu_out=relu_out),
        grid=(npad // block,),
        in_specs=[
            pl.BlockSpec((block, kin), lambda i: (i, 0)),
            pl.BlockSpec((kin, kout), lambda i: (0, 0)),
            pl.BlockSpec((1, kout), lambda i: (0, 0)),
        ],
        out_specs=pl.BlockSpec((block, kout), lambda i: (i, 0)),
        out_shape=jax.ShapeDtypeStruct((npad, kout), jnp.float32),
    )(x, w, b.reshape(1, kout))
    return out[:n]


def _edge_wmsg_body(gl_ref, gr_ref, ea_ref, we_ref, att_ref, o_ref,
                    *, H, C, HC, HCp):
    x = gl_ref[...] + gr_ref[...] + ea_ref[...] * we_ref[...]
    z = jnp.where(x >= 0, x, 0.2 * x)
    zw = z * att_ref[...]
    gl = gl_ref[...]
    tw = HCp - HC
    col = lax.broadcasted_iota(jnp.int32, (1, tw), 1)
    parts = []
    tail = jnp.zeros((gl.shape[0], tw), jnp.float32)
    for h in range(H):
        e = jnp.exp(jnp.sum(zw[:, h * C:(h + 1) * C], axis=1, keepdims=True))
        parts.append(e * gl[:, h * C:(h + 1) * C])
        tail = tail + jnp.where(col == h, e, 0.0)
    parts.append(tail)
    o_ref[...] = jnp.concatenate(parts, axis=1)


def _edge_wmsg(gl, gr, ea_col, we_row, att_row, H, C, HCp, block=1024):
    epad = gl.shape[0]
    hc = H * C
    return pl.pallas_call(
        functools.partial(_edge_wmsg_body, H=H, C=C, HC=hc, HCp=HCp),
        grid=(epad // block,),
        in_specs=[
            pl.BlockSpec((block, hc), lambda i: (i, 0)),
            pl.BlockSpec((block, hc), lambda i: (i, 0)),
            pl.BlockSpec((block, 1), lambda i: (i, 0)),
            pl.BlockSpec((1, hc), lambda i: (0, 0)),
            pl.BlockSpec((1, hc), lambda i: (0, 0)),
        ],
        out_specs=pl.BlockSpec((block, HCp), lambda i: (i, 0)),
        out_shape=jax.ShapeDtypeStruct((epad, HCp), jnp.float32),
    )(gl, gr, ea_col, we_row, att_row)


def _finalize_body(*refs, n_et, H, C, HC):
    o_ref = refs[-1]
    acc = None
    for i in range(n_et):
        agg = refs[i][...]
        b = refs[n_et + i][...]
        parts = []
        for h in range(H):
            s = agg[:, HC + h:HC + h + 1]
            parts.append(agg[:, h * C:(h + 1) * C] / (s + 1e-16))
        o = (jnp.concatenate(parts, axis=1) if H > 1 else parts[0]) + b
        acc = o if acc is None else acc + o
    o_ref[...] = acc / n_et


def _finalize(aggs, biases, nd, H, C, block=1024):
    n_et = len(aggs)
    hc = H * C
    hcp = aggs[0].shape[1]
    npad = aggs[0].shape[0]
    out = pl.pallas_call(
        functools.partial(_finalize_body, n_et=n_et, H=H, C=C, HC=hc),
        grid=(npad // block,),
        in_specs=[pl.BlockSpec((block, hcp), lambda i: (i, 0))] * n_et
        + [pl.BlockSpec((1, hc), lambda i: (0, 0))] * n_et,
        out_specs=pl.BlockSpec((block, hc), lambda i: (i, 0)),
        out_shape=jax.ShapeDtypeStruct((npad, hc), jnp.float32),
    )(*aggs, *[b.reshape(1, hc) for b in biases])
    return out[:nd]


def _gat_agg(p, x_s, x_d, ei, ea, H, C, relu_in, nd_pad):
    nd = x_d.shape[0]
    E = ea.shape[0]
    hc = H * C
    hcp = _ceil_to(hc + H, 32)
    xl = _linear(x_s, p['Wl'], p['bl'], relu_in=relu_in)
    xr = _linear(x_d, p['Wr'], p['br'], relu_in=relu_in)
    epad = _ceil_to(E, _SC_NC * _SC_NS * _SC_CH)
    src = jnp.pad(ei[0], (0, epad - E))
    dst = jnp.pad(ei[1], (0, epad - E), constant_values=nd)
    ea_col = jnp.pad(ea, (0, epad - E)).reshape(epad, 1)
    gl, gr = _sc_gather2(xl, src, xr, jnp.minimum(dst, nd - 1))
    wmsg = _edge_wmsg(gl, gr, ea_col, p['We'], p['att'].reshape(1, hc),
                      H, C, hcp)
    agg = jax.ops.segment_sum(wmsg.astype(jnp.bfloat16), dst,
                              num_segments=nd_pad).astype(jnp.float32)
    return agg


ETYPES_ = (('up', 'user', 'poi'), ('pu', 'poi', 'user'),
           ('pc', 'poi', 'category'), ('cp', 'category', 'poi'))


def kernel(x_user, x_poi, x_cat, e_up, e_pu, e_pc, e_cp,
           ea_up, ea_pu, ea_pc, ea_cp, params):
    enc = params['enc']
    x = {
        'user': _linear(x_user, enc['Wu'], enc['bu'], relu_out=True),
        'poi': _linear(x_poi, enc['Wp'], enc['bp'], relu_out=True),
        'category': _linear(x_cat, enc['Wc'], enc['bc'], relu_out=True)
                    + enc['emb'],
    }
    ei = {'up': e_up, 'pu': e_pu, 'pc': e_pc, 'cp': e_cp}
    ea = {'up': ea_up, 'pu': ea_pu, 'pc': ea_pc, 'cp': ea_cp}
    nnodes = {'user': x_user.shape[0], 'poi': x_poi.shape[0],
              'category': x_cat.shape[0]}
    nd_pads = {k: _ceil_to(v + 1, 2048) for k, v in nnodes.items()}

    def layer(conv, xin, H, C, relu_in):
        aggs = {}
        biases = {}
        for name, s, d in ETYPES_:
            a = _gat_agg(conv[name], xin[s], xin[d], ei[name], ea[name],
                         H, C, relu_in, nd_pads[d])
            aggs.setdefault(d, []).append(a)
            biases.setdefault(d, []).append(conv[name]['bias'])
        return {d: _finalize(aggs[d], biases[d], nnodes[d], H, C)
                for d in aggs}

    h = layer(params['conv1'], x, 2, 128, False)
    h = layer(params['conv2'], h, 1, 128, True)
    return (h['user'], h['poi'], h['category'])


# speedup vs baseline: 2.3154x; 2.3154x over previous
import functools

import jax
import jax.numpy as jnp
from jax import lax
from jax.experimental import pallas as pl
from jax.experimental.pallas import tpu as pltpu
from jax.experimental.pallas import tpu_sc as plsc


def _ceil_to(n, m):
    return (n + m - 1) // m * m


_SC_NC = 2
_SC_NS = 16
_SC_CH = 128


def _sc_gather(table, idx):
    n, d = table.shape
    e = idx.shape[0]
    nw = _SC_NC * _SC_NS
    bpw = e // nw
    nch = bpw // _SC_CH
    mesh = plsc.VectorSubcoreMesh(core_axis_name="c", subcore_axis_name="s")

    @functools.partial(
        pl.kernel, mesh=mesh,
        out_type=jax.ShapeDtypeStruct((e, d), jnp.float32),
        scratch_types=[
            pltpu.VMEM((_SC_CH,), jnp.int32),
            pltpu.VMEM((_SC_CH, d), jnp.float32),
            pltpu.SemaphoreType.DMA,
        ],
    )
    def gather_k(table_hbm, idx_hbm, out_hbm, idx_v, rows_v, sem):
        wid = lax.axis_index("s") * _SC_NC + lax.axis_index("c")
        base = wid * bpw

        def body(j, carry):
            st = base + j * _SC_CH
            pltpu.sync_copy(idx_hbm.at[pl.ds(st, _SC_CH)], idx_v)
            pltpu.async_copy(table_hbm.at[idx_v], rows_v, sem).wait()
            pltpu.sync_copy(rows_v, out_hbm.at[pl.ds(st, _SC_CH)])
            return carry

        lax.fori_loop(0, nch, body, 0)

    return gather_k(table, idx)


def _sc_gather2(table_l, idx_l, table_r, idx_r):
    n, d = table_l.shape
    e = idx_l.shape[0]
    nw = _SC_NC * _SC_NS
    bpw = e // nw
    nch = bpw // _SC_CH
    mesh = plsc.VectorSubcoreMesh(core_axis_name="c", subcore_axis_name="s")

    @functools.partial(
        pl.kernel, mesh=mesh,
        out_type=(jax.ShapeDtypeStruct((e, d), jnp.float32),
                  jax.ShapeDtypeStruct((e, d), jnp.float32)),
        scratch_types=[
            pltpu.VMEM((bpw,), jnp.int32),
            pltpu.VMEM((bpw,), jnp.int32),
            pltpu.VMEM((_SC_CH, d), jnp.float32),
            pltpu.VMEM((_SC_CH, d), jnp.float32),
            pltpu.SemaphoreType.DMA,
            pltpu.SemaphoreType.DMA,
        ],
    )
    def gather_k(tl_hbm, il_hbm, tr_hbm, ir_hbm, ol_hbm, or_hbm,
                 il_v, ir_v, rl_v, rr_v, sem_l, sem_r):
        wid = lax.axis_index("s") * _SC_NC + lax.axis_index("c")
        base = wid * bpw
        pltpu.sync_copy(il_hbm.at[pl.ds(base, bpw)], il_v)
        pltpu.sync_copy(ir_hbm.at[pl.ds(base, bpw)], ir_v)

        def body(j, carry):
            st = base + j * _SC_CH
            cl = pltpu.async_copy(tl_hbm.at[il_v.at[pl.ds(j * _SC_CH,
                                                          _SC_CH)]],
                                  rl_v, sem_l)
            cr = pltpu.async_copy(tr_hbm.at[ir_v.at[pl.ds(j * _SC_CH,
                                                          _SC_CH)]],
                                  rr_v, sem_r)
            cl.wait()
            pltpu.sync_copy(rl_v, ol_hbm.at[pl.ds(st, _SC_CH)])
            cr.wait()
            pltpu.sync_copy(rr_v, or_hbm.at[pl.ds(st, _SC_CH)])
            return carry

        lax.fori_loop(0, nch, body, 0)

    return gather_k(table_l, idx_l, table_r, idx_r)


def _sc_scatter(msg3, dst, nd_pad):
    nch, epad, w = msg3.shape
    nch2 = nch // _SC_NC
    ebt = epad // _SC_NS
    nech = ebt // _SC_CH
    rpt = nd_pad // _SC_NS
    zin = jnp.zeros((nd_pad, w), jnp.float32)
    mesh = plsc.VectorSubcoreMesh(core_axis_name="c", subcore_axis_name="s")

    @functools.partial(
        pl.kernel, mesh=mesh,
        out_type=jax.ShapeDtypeStruct((nch * nd_pad, w), jnp.float32),
        scratch_types=[
            pltpu.VMEM((_SC_CH,), jnp.int32),
            pltpu.VMEM((_SC_CH, w), jnp.float32),
            pltpu.VMEM_SHARED((nd_pad, w), jnp.float32),
        ],
    )
    def scat_k(msg_hbm, dst_hbm, zin_hbm, out_hbm, idx_v, rows_v, acc_sh):
        cid = lax.axis_index("c")
        sid = lax.axis_index("s")
        nrch = rpt // _SC_CH
        for c_local in range(nch2):
            c = cid * nch2 + c_local

            def zbody(k, carry):
                rs = sid * rpt + k * _SC_CH
                pltpu.sync_copy(zin_hbm.at[pl.ds(rs, _SC_CH)], rows_v)
                pltpu.sync_copy(rows_v, acc_sh.at[pl.ds(rs, _SC_CH)])
                return carry

            lax.fori_loop(0, nrch, zbody, 0)
            plsc.subcore_barrier()

            def body(j, carry):
                est = c * epad + sid * ebt + j * _SC_CH
                pltpu.sync_copy(dst_hbm.at[pl.ds(sid * ebt + j * _SC_CH,
                                                 _SC_CH)], idx_v)
                pltpu.sync_copy(msg_hbm.at[pl.ds(est, _SC_CH)], rows_v)
                pltpu.sync_copy(rows_v, acc_sh.at[idx_v], add=True)
                return carry

            lax.fori_loop(0, nech, body, 0)
            plsc.subcore_barrier()

            def wbody(k, carry):
                rs = sid * rpt + k * _SC_CH
                pltpu.sync_copy(acc_sh.at[pl.ds(rs, _SC_CH)], rows_v)
                pltpu.sync_copy(rows_v,
                                out_hbm.at[pl.ds(c * nd_pad + rs, _SC_CH)])
                return carry

            lax.fori_loop(0, nrch, wbody, 0)
            plsc.subcore_barrier()

    return scat_k(msg3.reshape(nch * epad, w), dst, zin).reshape(
        nch, nd_pad, w)


def _linear_body(x_ref, w_ref, b_ref, o_ref, *, relu_in, relu_out):
    x = x_ref[...]
    if relu_in:
        x = jnp.maximum(x, 0.0)
    y = jnp.dot(x, w_ref[...], preferred_element_type=jnp.float32) + b_ref[...]
    if relu_out:
        y = jnp.maximum(y, 0.0)
    o_ref[...] = y


def _linear(x, w, b, relu_in=False, relu_out=False, block=1024):
    n, kin = x.shape
    kout = w.shape[1]
    if kin < 8:
        x = jnp.pad(x, ((0, 0), (0, 8 - kin)))
        w = jnp.pad(w, ((0, 8 - kin), (0, 0)))
        kin = 8
    npad = _ceil_to(n, block)
    if npad != n:
        x = jnp.pad(x, ((0, npad - n), (0, 0)))
    out = pl.pallas_call(
        functools.partial(_linear_body, relu_in=relu_in, relu_out=relu_out),
        grid=(npad // block,),
        in_specs=[
            pl.BlockSpec((block, kin), lambda i: (i, 0)),
            pl.BlockSpec((kin, kout), lambda i: (0, 0)),
            pl.BlockSpec((1, kout), lambda i: (0, 0)),
        ],
        out_specs=pl.BlockSpec((block, kout), lambda i: (i, 0)),
        out_shape=jax.ShapeDtypeStruct((npad, kout), jnp.float32),
    )(x, w, b.reshape(1, kout))
    return out[:n]


def _edge_wmsg_body(gl_ref, gr_ref, ea_ref, we_ref, att_ref, o_ref,
                    *, H, C, HC, HCp):
    x = gl_ref[...] + gr_ref[...] + ea_ref[...] * we_ref[...]
    z = jnp.where(x >= 0, x, 0.2 * x)
    zw = z * att_ref[...]
    gl = gl_ref[...]
    tw = HCp - HC
    col = lax.broadcasted_iota(jnp.int32, (1, tw), 1)
    parts = []
    tail = jnp.zeros((gl.shape[0], tw), jnp.float32)
    for h in range(H):
        e = jnp.exp(jnp.sum(zw[:, h * C:(h + 1) * C], axis=1, keepdims=True))
        parts.append(e * gl[:, h * C:(h + 1) * C])
        tail = tail + jnp.where(col == h, e, 0.0)
    parts.append(tail)
    o_ref[...] = jnp.concatenate(parts, axis=1)


def _edge_wmsg(gl, gr, ea_col, we_row, att_row, H, C, HCp, block=1024):
    epad = gl.shape[0]
    hc = H * C
    return pl.pallas_call(
        functools.partial(_edge_wmsg_body, H=H, C=C, HC=hc, HCp=HCp),
        grid=(epad // block,),
        in_specs=[
            pl.BlockSpec((block, hc), lambda i: (i, 0)),
            pl.BlockSpec((block, hc), lambda i: (i, 0)),
            pl.BlockSpec((block, 1), lambda i: (i, 0)),
            pl.BlockSpec((1, hc), lambda i: (0, 0)),
            pl.BlockSpec((1, hc), lambda i: (0, 0)),
        ],
        out_specs=pl.BlockSpec((block, HCp), lambda i: (i, 0)),
        out_shape=jax.ShapeDtypeStruct((epad, HCp), jnp.float32),
    )(gl, gr, ea_col, we_row, att_row)


def _finalize_body(*refs, n_et, H, C, HC):
    o_ref = refs[-1]
    acc = None
    for i in range(n_et):
        agg = refs[i][...]
        b = refs[n_et + i][...]
        parts = []
        for h in range(H):
            s = agg[:, HC + h:HC + h + 1]
            parts.append(agg[:, h * C:(h + 1) * C] / (s + 1e-16))
        o = (jnp.concatenate(parts, axis=1) if H > 1 else parts[0]) + b
        acc = o if acc is None else acc + o
    o_ref[...] = acc / n_et


def _finalize(aggs, biases, nd, H, C, block=1024):
    n_et = len(aggs)
    hc = H * C
    hcp = aggs[0].shape[1]
    npad = aggs[0].shape[0]
    out = pl.pallas_call(
        functools.partial(_finalize_body, n_et=n_et, H=H, C=C, HC=hc),
        grid=(npad // block,),
        in_specs=[pl.BlockSpec((block, hcp), lambda i: (i, 0))] * n_et
        + [pl.BlockSpec((1, hc), lambda i: (0, 0))] * n_et,
        out_specs=pl.BlockSpec((block, hc), lambda i: (i, 0)),
        out_shape=jax.ShapeDtypeStruct((npad, hc), jnp.float32),
    )(*aggs, *[b.reshape(1, hc) for b in biases])
    return out[:nd]


def _gat_agg(p, x_s, x_d, ei, ea, H, C, relu_in, nd_pad):
    nd = x_d.shape[0]
    E = ea.shape[0]
    hc = H * C
    hcp = _ceil_to(hc + H, 32)
    xl = _linear(x_s, p['Wl'], p['bl'], relu_in=relu_in)
    xr = _linear(x_d, p['Wr'], p['br'], relu_in=relu_in)
    epad = _ceil_to(E, _SC_NC * _SC_NS * _SC_CH)
    src = jnp.pad(ei[0], (0, epad - E))
    dst = jnp.pad(ei[1], (0, epad - E), constant_values=nd)
    ea_col = jnp.pad(ea, (0, epad - E)).reshape(epad, 1)
    gl, gr = _sc_gather2(xl, src, xr, jnp.minimum(dst, nd - 1))
    wmsg = _edge_wmsg(gl, gr, ea_col, p['We'], p['att'].reshape(1, hc),
                      H, C, hcp)
    agg = jax.ops.segment_sum(wmsg, dst, num_segments=nd_pad)
    return agg


ETYPES_ = (('up', 'user', 'poi'), ('pu', 'poi', 'user'),
           ('pc', 'poi', 'category'), ('cp', 'category', 'poi'))


def kernel(x_user, x_poi, x_cat, e_up, e_pu, e_pc, e_cp,
           ea_up, ea_pu, ea_pc, ea_cp, params):
    enc = params['enc']
    x = {
        'user': _linear(x_user, enc['Wu'], enc['bu'], relu_out=True),
        'poi': _linear(x_poi, enc['Wp'], enc['bp'], relu_out=True),
        'category': _linear(x_cat, enc['Wc'], enc['bc'], relu_out=True)
                    + enc['emb'],
    }
    ei = {'up': e_up, 'pu': e_pu, 'pc': e_pc, 'cp': e_cp}
    ea = {'up': ea_up, 'pu': ea_pu, 'pc': ea_pc, 'cp': ea_cp}
    nnodes = {'user': x_user.shape[0], 'poi': x_poi.shape[0],
              'category': x_cat.shape[0]}
    nd_pads = {k: _ceil_to(v + 1, 2048) for k, v in nnodes.items()}

    def layer(conv, xin, H, C, relu_in):
        aggs = {}
        biases = {}
        for name, s, d in ETYPES_:
            a = _gat_agg(conv[name], xin[s], xin[d], ei[name], ea[name],
                         H, C, relu_in, nd_pads[d])
            aggs.setdefault(d, []).append(a)
            biases.setdefault(d, []).append(conv[name]['bias'])
        return {d: _finalize(aggs[d], biases[d], nnodes[d], H, C)
                for d in aggs}

    h = layer(params['conv1'], x, 2, 128, False)
    h = layer(params['conv2'], h, 1, 128, True)
    return (h['user'], h['poi'], h['category'])


# speedup vs baseline: 2.3184x; 1.0013x over previous
import functools

import jax
import jax.numpy as jnp
from jax import lax
from jax.experimental import pallas as pl
from jax.experimental.pallas import tpu as pltpu
from jax.experimental.pallas import tpu_sc as plsc


def _ceil_to(n, m):
    return (n + m - 1) // m * m


_SC_NC = 2
_SC_NS = 16
_SC_CH = 128


def _sc_gather2(table_l, idx_l, table_r, idx_r):
    n, d = table_l.shape
    e = idx_l.shape[0]
    nw = _SC_NC * _SC_NS
    bpw = e // nw
    nch = bpw // _SC_CH
    mesh = plsc.VectorSubcoreMesh(core_axis_name="c", subcore_axis_name="s")

    @functools.partial(
        pl.kernel, mesh=mesh,
        out_type=(jax.ShapeDtypeStruct((e, d), jnp.float32),
                  jax.ShapeDtypeStruct((e, d), jnp.float32)),
        scratch_types=[
            pltpu.VMEM((bpw,), jnp.int32),
            pltpu.VMEM((bpw,), jnp.int32),
            pltpu.VMEM((_SC_CH, d), jnp.float32),
            pltpu.VMEM((_SC_CH, d), jnp.float32),
            pltpu.SemaphoreType.DMA,
            pltpu.SemaphoreType.DMA,
        ],
    )
    def gather_k(tl_hbm, il_hbm, tr_hbm, ir_hbm, ol_hbm, or_hbm,
                 il_v, ir_v, rl_v, rr_v, sem_l, sem_r):
        wid = lax.axis_index("s") * _SC_NC + lax.axis_index("c")
        base = wid * bpw
        pltpu.sync_copy(il_hbm.at[pl.ds(base, bpw)], il_v)
        pltpu.sync_copy(ir_hbm.at[pl.ds(base, bpw)], ir_v)

        def body(j, carry):
            st = base + j * _SC_CH
            cl = pltpu.async_copy(tl_hbm.at[il_v.at[pl.ds(j * _SC_CH,
                                                          _SC_CH)]],
                                  rl_v, sem_l)
            cr = pltpu.async_copy(tr_hbm.at[ir_v.at[pl.ds(j * _SC_CH,
                                                          _SC_CH)]],
                                  rr_v, sem_r)
            cl.wait()
            pltpu.sync_copy(rl_v, ol_hbm.at[pl.ds(st, _SC_CH)])
            cr.wait()
            pltpu.sync_copy(rr_v, or_hbm.at[pl.ds(st, _SC_CH)])
            return carry

        lax.fori_loop(0, nch, body, 0)

    return gather_k(table_l, idx_l, table_r, idx_r)


def _linear_body(x_ref, w_ref, b_ref, o_ref, *, relu_in, relu_out):
    x = x_ref[...]
    if relu_in:
        x = jnp.maximum(x, 0.0)
    y = jnp.dot(x, w_ref[...], preferred_element_type=jnp.float32) + b_ref[...]
    if relu_out:
        y = jnp.maximum(y, 0.0)
    o_ref[...] = y


def _linear(x, w, b, relu_in=False, relu_out=False, block=1024):
    n, kin = x.shape
    kout = w.shape[1]
    if kin < 8:
        x = jnp.pad(x, ((0, 0), (0, 8 - kin)))
        w = jnp.pad(w, ((0, 8 - kin), (0, 0)))
        kin = 8
    npad = _ceil_to(n, block)
    if npad != n:
        x = jnp.pad(x, ((0, npad - n), (0, 0)))
    out = pl.pallas_call(
        functools.partial(_linear_body, relu_in=relu_in, relu_out=relu_out),
        grid=(npad // block,),
        in_specs=[
            pl.BlockSpec((block, kin), lambda i: (i, 0)),
            pl.BlockSpec((kin, kout), lambda i: (0, 0)),
            pl.BlockSpec((1, kout), lambda i: (0, 0)),
        ],
        out_specs=pl.BlockSpec((block, kout), lambda i: (i, 0)),
        out_shape=jax.ShapeDtypeStruct((npad, kout), jnp.float32),
    )(x, w, b.reshape(1, kout))
    return out[:n]


def _edge_wmsg_body(gl_ref, gr_ref, ea_ref, we_ref, att_ref, o_ref,
                    *, H, C, HC, HCp):
    x = gl_ref[...] + gr_ref[...] + ea_ref[...] * we_ref[...]
    z = jnp.where(x >= 0, x, 0.2 * x)
    zw = z * att_ref[...]
    gl = gl_ref[...]
    tw = HCp - HC
    col = lax.broadcasted_iota(jnp.int32, (1, tw), 1)
    parts = []
    tail = jnp.zeros((gl.shape[0], tw), jnp.float32)
    for h in range(H):
        e = jnp.exp(jnp.sum(zw[:, h * C:(h + 1) * C], axis=1, keepdims=True))
        parts.append(e * gl[:, h * C:(h + 1) * C])
        tail = tail + jnp.where(col == h, e, 0.0)
    parts.append(tail)
    o_ref[...] = jnp.concatenate(parts, axis=1)


def _edge_wmsg(gl, gr, ea_col, we_row, att_row, H, C, HCp, block=1024):
    epad = gl.shape[0]
    hc = H * C
    return pl.pallas_call(
        functools.partial(_edge_wmsg_body, H=H, C=C, HC=hc, HCp=HCp),
        grid=(epad // block,),
        in_specs=[
            pl.BlockSpec((block, hc), lambda i: (i, 0)),
            pl.BlockSpec((block, hc), lambda i: (i, 0)),
            pl.BlockSpec((block, 1), lambda i: (i, 0)),
            pl.BlockSpec((1, hc), lambda i: (0, 0)),
            pl.BlockSpec((1, hc), lambda i: (0, 0)),
        ],
        out_specs=pl.BlockSpec((block, HCp), lambda i: (i, 0)),
        out_shape=jax.ShapeDtypeStruct((epad, HCp), jnp.float32),
    )(gl, gr, ea_col, we_row, att_row)


def _finalize_body(*refs, n_et, H, C, HC):
    o_ref = refs[-1]
    acc = None
    for i in range(n_et):
        agg = refs[i][...]
        b = refs[n_et + i][...]
        parts = []
        for h in range(H):
            s = agg[:, HC + h:HC + h + 1]
            parts.append(agg[:, h * C:(h + 1) * C] / (s + 1e-16))
        o = (jnp.concatenate(parts, axis=1) if H > 1 else parts[0]) + b
        acc = o if acc is None else acc + o
    o_ref[...] = acc / n_et


def _finalize(aggs, biases, nd, H, C, block=1024):
    n_et = len(aggs)
    hc = H * C
    hcp = aggs[0].shape[1]
    npad = aggs[0].shape[0]
    out = pl.pallas_call(
        functools.partial(_finalize_body, n_et=n_et, H=H, C=C, HC=hc),
        grid=(npad // block,),
        in_specs=[pl.BlockSpec((block, hcp), lambda i: (i, 0))] * n_et
        + [pl.BlockSpec((1, hc), lambda i: (0, 0))] * n_et,
        out_specs=pl.BlockSpec((block, hc), lambda i: (i, 0)),
        out_shape=jax.ShapeDtypeStruct((npad, hc), jnp.float32),
    )(*aggs, *[b.reshape(1, hc) for b in biases])
    return out[:nd]


def _gat_agg(p, x_s, x_d, ei, ea, H, C, relu_in, nd_pad):
    nd = x_d.shape[0]
    E = ea.shape[0]
    hc = H * C
    hcp = _ceil_to(hc + H, 32)
    xl = _linear(x_s, p['Wl'], p['bl'], relu_in=relu_in)
    xr = _linear(x_d, p['Wr'], p['br'], relu_in=relu_in)
    epad = _ceil_to(E, _SC_NC * _SC_NS * _SC_CH)
    src = jnp.pad(ei[0], (0, epad - E))
    dst = jnp.pad(ei[1], (0, epad - E), constant_values=nd)
    ea_col = jnp.pad(ea, (0, epad - E)).reshape(epad, 1)
    gl, gr = _sc_gather2(xl, src, xr, jnp.minimum(dst, nd - 1))
    wmsg = _edge_wmsg(gl, gr, ea_col, p['We'], p['att'].reshape(1, hc),
                      H, C, hcp)
    agg = jax.ops.segment_sum(wmsg, dst, num_segments=nd_pad)
    return agg


ETYPES_ = (('up', 'user', 'poi'), ('pu', 'poi', 'user'),
           ('pc', 'poi', 'category'), ('cp', 'category', 'poi'))


def kernel(x_user, x_poi, x_cat, e_up, e_pu, e_pc, e_cp,
           ea_up, ea_pu, ea_pc, ea_cp, params):
    enc = params['enc']
    x = {
        'user': _linear(x_user, enc['Wu'], enc['bu'], relu_out=True),
        'poi': _linear(x_poi, enc['Wp'], enc['bp'], relu_out=True),
        'category': _linear(x_cat, enc['Wc'], enc['bc'], relu_out=True)
                    + enc['emb'],
    }
    ei = {'up': e_up, 'pu': e_pu, 'pc': e_pc, 'cp': e_cp}
    ea = {'up': ea_up, 'pu': ea_pu, 'pc': ea_pc, 'cp': ea_cp}
    nnodes = {'user': x_user.shape[0], 'poi': x_poi.shape[0],
              'category': x_cat.shape[0]}
    nd_pads = {k: _ceil_to(v + 1, 2048) for k, v in nnodes.items()}

    def layer(conv, xin, H, C, relu_in):
        aggs = {}
        biases = {}
        for name, s, d in ETYPES_:
            a = _gat_agg(conv[name], xin[s], xin[d], ei[name], ea[name],
                         H, C, relu_in, nd_pads[d])
            aggs.setdefault(d, []).append(a)
            biases.setdefault(d, []).append(conv[name]['bias'])
        return {d: _finalize(aggs[d], biases[d], nnodes[d], H, C)
                for d in aggs}

    h = layer(params['conv1'], x, 2, 128, False)
    h = layer(params['conv2'], h, 1, 128, True)
    return (h['user'], h['poi'], h['category'])
